# baseline (speedup 1.0000x reference)
import functools

import jax
import jax.numpy as jnp
from jax import lax
from jax.experimental import pallas as pl
from jax.experimental.pallas import tpu as pltpu
from jax.experimental.pallas import tpu_sc as plsc

N_NODES = 1000000
EMB_DIM = 64
BATCH = 16384

_NC = 2
_NS = 16
_NW = _NC * _NS
_L = 16
_BPW = BATCH // _NW
_HB = _BPW // 2
_R2 = N_NODES // 2

_GD = jax.lax.GatherDimensionNumbers(
    offset_dims=(), collapsed_slice_dims=(0,), start_index_map=(0,))


def _bcast(ref_vec, pos):
    vb = pl.multiple_of((pos >> 4) << 4, 16)
    v = ref_vec[pl.ds(vb, _L)]
    return jax.lax.gather(
        v, jnp.full((_L, 1), pos & 15, jnp.int32), _GD, (1,),
        mode=jax.lax.GatherScatterMode.PROMISE_IN_BOUNDS)


def _body(sh_h, th_h, sp_h, tp_h, src_raw, dst_raw, rel_h, bu_h, bv_h, out_h,
          idx_s, idx_t, par_s, par_t, u2, v2, rel_v, bs_v, bt_v, out_v,
          sem_u, sem_v, sem_bs, sem_bt):
    src2 = src_raw
    dst2 = dst_raw
    w = lax.axis_index("s") * _NC + lax.axis_index("c")
    basep = w * _BPW
    iota = lax.iota(jnp.int32, _L)
    mask15 = iota == (_L - 1)

    pltpu.sync_copy(sh_h.at[pl.ds(basep, _BPW)], idx_s)
    pltpu.sync_copy(th_h.at[pl.ds(basep, _BPW)], idx_t)
    pltpu.sync_copy(sp_h.at[pl.ds(basep, _BPW)], par_s)
    pltpu.sync_copy(tp_h.at[pl.ds(basep, _BPW)], par_t)
    pltpu.sync_copy(rel_h, rel_v)
    cbs = pltpu.async_copy(bu_h.at[idx_s], bs_v, sem_bs)
    cbt = pltpu.async_copy(bv_h.at[idx_t], bt_v, sem_bt)

    r = [rel_v[pl.ds(q * _L, _L)] for q in range(4)]

    for half in range(2):
        hb = half * _HB
        cu = pltpu.async_copy(
            src2.at[idx_s.at[pl.ds(hb, _HB)]], u2, sem_u)
        cv = pltpu.async_copy(
            dst2.at[idx_t.at[pl.ds(hb, _HB)]], v2, sem_v)
        cu.wait()
        cv.wait()

        def row(i, carry, hb=hb):
            pu = _bcast(par_s, hb + i) > 0
            pv = _bcast(par_t, hb + i) > 0
            acc = None
            for q in range(4):
                ua = u2[i, pl.ds(q * _L, _L)]
                ub = u2[i, pl.ds(EMB_DIM + q * _L, _L)]
                va = v2[i, pl.ds(q * _L, _L)]
                vb = v2[i, pl.ds(EMB_DIM + q * _L, _L)]
                uq = jnp.where(pu, ub, ua)
                vq = jnp.where(pv, vb, va)
                term = uq * r[q] * vq
                acc = term if acc is None else acc + term
            c = plsc.cumsum(acc)
            plsc.store_scatter(
                out_v, [jnp.full((_L,), hb + i, jnp.int32)], c, mask=mask15)
            return carry

        lax.fori_loop(0, _HB, row, 0)

    cbs.wait()
    cbt.wait()
    for blk in range(_BPW // _L):
        sl = pl.ds(blk * _L, _L)
        out_v[sl] = out_v[sl] + bs_v[sl] + bt_v[sl]
    pltpu.sync_copy(out_v, out_h.at[pl.ds(basep, _BPW)])


@jax.jit
def _run(pairs, src, dst, rel, bu, bv):
    s = pairs[:, 0].astype(jnp.int32)
    t = pairs[:, 1].astype(jnp.int32)
    sh = s >> 1
    th = t >> 1
    sp = (s & 1) * EMB_DIM
    tp = (t & 1) * EMB_DIM
    srcw = src.reshape(_R2, 2 * EMB_DIM)
    dstw = dst.reshape(_R2, 2 * EMB_DIM)

    mesh = plsc.VectorSubcoreMesh(core_axis_name="c", subcore_axis_name="s")
    kern = functools.partial(
        pl.kernel,
        mesh=mesh,
        compiler_params=pltpu.CompilerParams(
            needs_layout_passes=False, use_tc_tiling_on_sc=True),
        out_type=jax.ShapeDtypeStruct((BATCH,), jnp.float32),
        scratch_types=[
            pltpu.VMEM((_BPW,), jnp.int32),
            pltpu.VMEM((_BPW,), jnp.int32),
            pltpu.VMEM((_BPW,), jnp.int32),
            pltpu.VMEM((_BPW,), jnp.int32),
            pltpu.VMEM((_HB, 2 * EMB_DIM), jnp.float32),
            pltpu.VMEM((_HB, 2 * EMB_DIM), jnp.float32),
            pltpu.VMEM((EMB_DIM,), jnp.float32),
            pltpu.VMEM((_BPW,), jnp.float32),
            pltpu.VMEM((_BPW,), jnp.float32),
            pltpu.VMEM((_BPW,), jnp.float32),
            pltpu.SemaphoreType.DMA,
            pltpu.SemaphoreType.DMA,
            pltpu.SemaphoreType.DMA,
            pltpu.SemaphoreType.DMA,
        ],
    )(_body)
    return kern(sh, th, sp, tp, srcw, dstw, rel,
                bu.reshape(N_NODES), bv.reshape(N_NODES))


def kernel(pairs, src, dst, rel, bu, bv):
    return _run(pairs, src, dst, rel, bu, bv)


# speedup vs baseline: 1.0720x; 1.0720x over previous
import functools

import jax
import jax.numpy as jnp
from jax import lax
from jax.experimental import pallas as pl
from jax.experimental.pallas import tpu as pltpu
from jax.experimental.pallas import tpu_sc as plsc

N_NODES = 1000000
EMB_DIM = 64
BATCH = 16384

_NC = 2
_NS = 16
_NW = _NC * _NS
_L = 16
_BPW = BATCH // _NW
_HB = _BPW // 2


def _body(s_h, t_h, srcw, dstw, rel_h, bu_h, bv_h, out_h,
          idx_s, idx_t, u2, v2, rel_v, bs_v, bt_v, out_v,
          sem_u, sem_v, sem_bs, sem_bt):
    w = lax.axis_index("s") * _NC + lax.axis_index("c")
    basep = w * _BPW
    iota = lax.iota(jnp.int32, _L)
    mask15 = iota == (_L - 1)

    pltpu.sync_copy(s_h.at[pl.ds(basep, _BPW)], idx_s)
    pltpu.sync_copy(t_h.at[pl.ds(basep, _BPW)], idx_t)
    pltpu.sync_copy(rel_h, rel_v)
    cbs = pltpu.async_copy(bu_h.at[idx_s], bs_v, sem_bs)
    cbt = pltpu.async_copy(bv_h.at[idx_t], bt_v, sem_bt)

    r = [rel_v[pl.ds(q * _L, _L)] for q in range(4)]

    for half in range(2):
        hb = half * _HB
        cu = pltpu.async_copy(
            srcw.at[idx_s.at[pl.ds(hb, _HB)]], u2, sem_u)
        cv = pltpu.async_copy(
            dstw.at[idx_t.at[pl.ds(hb, _HB)]], v2, sem_v)
        cu.wait()
        cv.wait()

        def row(i, carry, hb=hb):
            acc = None
            for q in range(4):
                uq = u2[i, pl.ds(q * _L, _L)]
                vq = v2[i, pl.ds(q * _L, _L)]
                term = uq * r[q] * vq
                acc = term if acc is None else acc + term
            c = plsc.cumsum(acc)
            plsc.store_scatter(
                out_v, [jnp.full((_L,), hb + i, jnp.int32)], c, mask=mask15)
            return carry

        lax.fori_loop(0, _HB, row, 0)

    cbs.wait()
    cbt.wait()
    for blk in range(_BPW // _L):
        sl = pl.ds(blk * _L, _L)
        out_v[sl] = out_v[sl] + bs_v[sl] + bt_v[sl]
    pltpu.sync_copy(out_v, out_h.at[pl.ds(basep, _BPW)])


@jax.jit
def _run(pairs, src, dst, rel, bu, bv):
    s = pairs[:, 0].astype(jnp.int32)
    t = pairs[:, 1].astype(jnp.int32)
    srcw = jnp.pad(src, ((0, 0), (0, EMB_DIM)))
    dstw = jnp.pad(dst, ((0, 0), (0, EMB_DIM)))

    mesh = plsc.VectorSubcoreMesh(core_axis_name="c", subcore_axis_name="s")
    kern = functools.partial(
        pl.kernel,
        mesh=mesh,
        compiler_params=pltpu.CompilerParams(
            needs_layout_passes=False, use_tc_tiling_on_sc=True),
        out_type=jax.ShapeDtypeStruct((BATCH,), jnp.float32),
        scratch_types=[
            pltpu.VMEM((_BPW,), jnp.int32),
            pltpu.VMEM((_BPW,), jnp.int32),
            pltpu.VMEM((_HB, 2 * EMB_DIM), jnp.float32),
            pltpu.VMEM((_HB, 2 * EMB_DIM), jnp.float32),
            pltpu.VMEM((EMB_DIM,), jnp.float32),
            pltpu.VMEM((_BPW,), jnp.float32),
            pltpu.VMEM((_BPW,), jnp.float32),
            pltpu.VMEM((_BPW,), jnp.float32),
            pltpu.SemaphoreType.DMA,
            pltpu.SemaphoreType.DMA,
            pltpu.SemaphoreType.DMA,
            pltpu.SemaphoreType.DMA,
        ],
    )(_body)
    return kern(s, t, srcw, dstw, rel,
                bu.reshape(N_NODES), bv.reshape(N_NODES))


def kernel(pairs, src, dst, rel, bu, bv):
    return _run(pairs, src, dst, rel, bu, bv)
